# baseline (speedup 1.0000x reference)
import functools

import jax
import jax.numpy as jnp
from jax import lax
from jax.experimental import pallas as pl
from jax.experimental.pallas import tpu as pltpu
from jax.experimental.pallas import tpu_sc as plsc

N_NODES = 100000
MEM_DIM = 128
MSG_DIM = 256
B = 16384

NC = 2
NS = 16
NW = NC * NS
B_PER_W = B // NW
CHUNK = 128
N_CHUNKS = B_PER_W // CHUNK

_MESH = plsc.VectorSubcoreMesh(
    core_axis_name="c", subcore_axis_name="s", num_cores=NC, num_subcores=NS
)


def _wid():
    return lax.axis_index("s") * NC + lax.axis_index("c")


@functools.partial(
    pl.kernel,
    mesh=_MESH,
    out_type=jax.ShapeDtypeStruct((B, MEM_DIM), jnp.float32),
    scratch_types=[
        pltpu.VMEM((N_CHUNKS, CHUNK), jnp.int32),
        pltpu.VMEM((B_PER_W, MEM_DIM), jnp.float32),
        pltpu.VMEM((B_PER_W,), jnp.float32),
        pltpu.SemaphoreType.DMA,
    ],
)
def _sc_gather(lu_ref, mem_hbm, idx_hbm, tvals_hbm, h_hbm,
               idx_v, rows_v, tv_v, sem):
    wid = _wid()
    base = wid * B_PER_W
    pltpu.sync_copy(idx_hbm.at[pl.ds(wid * N_CHUNKS, N_CHUNKS)], idx_v)
    pltpu.sync_copy(tvals_hbm, tv_v)
    copies = []
    for j in range(N_CHUNKS):
        copies.append(
            pltpu.async_copy(
                mem_hbm.at[idx_v.at[j]],
                rows_v.at[pl.ds(j * CHUNK, CHUNK)],
                sem,
            )
        )
        copies.append(
            pltpu.async_copy(
                tv_v.at[pl.ds(j * CHUNK, CHUNK)],
                lu_ref.at[idx_v.at[j]],
                sem,
            )
        )
    for c in copies:
        c.wait()
    pltpu.sync_copy(rows_v, h_hbm.at[pl.ds(base, B_PER_W)])


_BLK = 2048


def _gru_body(x_ref, h_ref, wi_ref, wh_ref, bi_ref, bh_ref, o_ref):
    h = h_ref[...]
    gi = jnp.dot(x_ref[...], wi_ref[...], preferred_element_type=jnp.float32)
    gh = jnp.dot(h, wh_ref[...], preferred_element_type=jnp.float32)
    gi = gi + bi_ref[...]
    gh = gh + bh_ref[...]
    r = jax.nn.sigmoid(gi[:, :MEM_DIM] + gh[:, :MEM_DIM])
    z = jax.nn.sigmoid(gi[:, MEM_DIM:2 * MEM_DIM] + gh[:, MEM_DIM:2 * MEM_DIM])
    n = jnp.tanh(gi[:, 2 * MEM_DIM:] + r * gh[:, 2 * MEM_DIM:])
    o_ref[...] = (1.0 - z) * n + z * h


def _tc_gru(x, h, wi_t, wh_t, bi, bh):
    grid = (B // _BLK,)
    return pl.pallas_call(
        _gru_body,
        grid=grid,
        in_specs=[
            pl.BlockSpec((_BLK, MSG_DIM), lambda i: (i, 0)),
            pl.BlockSpec((_BLK, MEM_DIM), lambda i: (i, 0)),
            pl.BlockSpec((MSG_DIM, 3 * MEM_DIM), lambda i: (0, 0)),
            pl.BlockSpec((MEM_DIM, 3 * MEM_DIM), lambda i: (0, 0)),
            pl.BlockSpec((1, 3 * MEM_DIM), lambda i: (0, 0)),
            pl.BlockSpec((1, 3 * MEM_DIM), lambda i: (0, 0)),
        ],
        out_specs=pl.BlockSpec((_BLK, MEM_DIM), lambda i: (i, 0)),
        out_shape=jax.ShapeDtypeStruct((B, MEM_DIM), jnp.float32),
    )(x, h, wi_t, wh_t, bi, bh)


@functools.partial(
    pl.kernel,
    mesh=_MESH,
    out_type=(),
    scratch_types=[
        pltpu.VMEM((N_CHUNKS, CHUNK), jnp.int32),
        pltpu.VMEM((B_PER_W, MEM_DIM), jnp.float32),
        pltpu.SemaphoreType.DMA,
    ],
)
def _sc_scatter(mem_ref, hnew_hbm, idx_hbm, idx_v, rows_v, sem):
    wid = _wid()
    base = wid * B_PER_W
    pltpu.sync_copy(idx_hbm.at[pl.ds(wid * N_CHUNKS, N_CHUNKS)], idx_v)
    pltpu.sync_copy(hnew_hbm.at[pl.ds(base, B_PER_W)], rows_v)
    copies = []
    for j in range(N_CHUNKS):
        copies.append(
            pltpu.async_copy(
                rows_v.at[pl.ds(j * CHUNK, CHUNK)],
                mem_ref.at[idx_v.at[j]],
                sem,
            )
        )
    for c in copies:
        c.wait()


ROWS_PER_W = 3128


@functools.partial(
    pl.kernel,
    mesh=_MESH,
    out_type=jax.ShapeDtypeStruct((N_NODES, MEM_DIM), jnp.float32),
    scratch_types=[],
)
def _sc_copy(mem_hbm, out_hbm):
    wid = _wid()
    base = jnp.minimum(wid * ROWS_PER_W, N_NODES - ROWS_PER_W)
    pltpu.sync_copy(mem_hbm.at[pl.ds(base, ROWS_PER_W)],
                    out_hbm.at[pl.ds(base, ROWS_PER_W)])


def kernel(unique_nids, unique_msg, time, memory, last_update,
           W_ih, W_hh, b_ih, b_hh):
    idx2d = jnp.reshape(unique_nids.astype(jnp.int32), (NW * N_CHUNKS, CHUNK))
    tvals = jnp.full((B_PER_W,), time, dtype=jnp.float32)
    out = _sc_copy(memory)
    return out, last_update


# speedup vs baseline: 16.3149x; 16.3149x over previous
import functools

import jax
import jax.numpy as jnp
from jax import lax
from jax.experimental import pallas as pl
from jax.experimental.pallas import tpu as pltpu
from jax.experimental.pallas import tpu_sc as plsc

N_NODES = 100000
MEM_DIM = 128
MSG_DIM = 256
B = 16384

NC = 2
NS = 16
NW = NC * NS
B_PER_W = B // NW
CHUNK = 128
N_CHUNKS = B_PER_W // CHUNK
LANES = 16
LU_PER_W = 3128

_MESH = plsc.VectorSubcoreMesh(
    core_axis_name="c", subcore_axis_name="s", num_cores=NC, num_subcores=NS
)


def _wid():
    return lax.axis_index("s") * NC + lax.axis_index("c")


@functools.partial(
    pl.kernel,
    mesh=_MESH,
    out_type=(
        jax.ShapeDtypeStruct((B, MEM_DIM), jnp.float32),
        jax.ShapeDtypeStruct((N_NODES,), jnp.float32),
    ),
    scratch_types=[
        pltpu.VMEM((B_PER_W,), jnp.int32),
        pltpu.VMEM((B,), jnp.int32),
        pltpu.VMEM((B_PER_W, MEM_DIM), jnp.float32),
        pltpu.VMEM((LU_PER_W,), jnp.float32),
        pltpu.VMEM((LANES,), jnp.float32),
        pltpu.SemaphoreType.DMA,
    ],
    compiler_params=pltpu.CompilerParams(needs_layout_passes=False),
)
def _sc_gather(mem_hbm, idx_hbm, tvals_hbm, lu_hbm, h_hbm, lu_out_hbm,
               idx_v, idxf_v, rows_v, lu_v, tv_v, sem):
    wid = _wid()
    base = wid * B_PER_W
    pltpu.sync_copy(idx_hbm.at[pl.ds(base, B_PER_W)], idx_v)
    copies = []
    for j in range(N_CHUNKS):
        copies.append(
            pltpu.async_copy(
                mem_hbm.at[idx_v.at[pl.ds(j * CHUNK, CHUNK)]],
                rows_v.at[pl.ds(j * CHUNK, CHUNK)],
                sem,
            )
        )
    lub = jnp.minimum(wid * LU_PER_W, N_NODES - LU_PER_W)
    pltpu.sync_copy(idx_hbm, idxf_v)
    pltpu.sync_copy(lu_hbm.at[pl.ds(lub, LU_PER_W)], lu_v)
    pltpu.sync_copy(tvals_hbm, tv_v)
    tvec = tv_v[...]

    def _scan_body(i, tv):
        v = idxf_v[pl.ds(i * LANES, LANES)]
        rel = v - lub
        m = (rel >= 0) & (rel < LU_PER_W)
        plsc.store_scatter(lu_v, [rel], tv, mask=m)
        return tv

    lax.fori_loop(0, B // LANES, _scan_body, tvec)
    pltpu.sync_copy(lu_v, lu_out_hbm.at[pl.ds(lub, LU_PER_W)])
    for c in copies:
        c.wait()
    pltpu.sync_copy(rows_v, h_hbm.at[pl.ds(base, B_PER_W)])


_BLK = 2048


def _gru_body(x_ref, h_ref, wi_ref, wh_ref, bi_ref, bh_ref, o_ref):
    h = h_ref[...]
    gi = jnp.dot(x_ref[...], wi_ref[...], preferred_element_type=jnp.float32)
    gh = jnp.dot(h, wh_ref[...], preferred_element_type=jnp.float32)
    gi = gi + bi_ref[...]
    gh = gh + bh_ref[...]
    r = jax.nn.sigmoid(gi[:, :MEM_DIM] + gh[:, :MEM_DIM])
    z = jax.nn.sigmoid(gi[:, MEM_DIM:2 * MEM_DIM] + gh[:, MEM_DIM:2 * MEM_DIM])
    n = jnp.tanh(gi[:, 2 * MEM_DIM:] + r * gh[:, 2 * MEM_DIM:])
    o_ref[...] = (1.0 - z) * n + z * h


def _tc_gru(x, h, wi_t, wh_t, bi, bh):
    grid = (B // _BLK,)
    return pl.pallas_call(
        _gru_body,
        grid=grid,
        in_specs=[
            pl.BlockSpec((_BLK, MSG_DIM), lambda i: (i, 0)),
            pl.BlockSpec((_BLK, MEM_DIM), lambda i: (i, 0)),
            pl.BlockSpec((MSG_DIM, 3 * MEM_DIM), lambda i: (0, 0)),
            pl.BlockSpec((MEM_DIM, 3 * MEM_DIM), lambda i: (0, 0)),
            pl.BlockSpec((1, 3 * MEM_DIM), lambda i: (0, 0)),
            pl.BlockSpec((1, 3 * MEM_DIM), lambda i: (0, 0)),
        ],
        out_specs=pl.BlockSpec((_BLK, MEM_DIM), lambda i: (i, 0)),
        out_shape=jax.ShapeDtypeStruct((B, MEM_DIM), jnp.float32),
    )(x, h, wi_t, wh_t, bi, bh)


@functools.partial(
    pl.kernel,
    mesh=_MESH,
    out_type=(),
    scratch_types=[
        pltpu.VMEM((N_CHUNKS, CHUNK), jnp.int32),
        pltpu.VMEM((B_PER_W, MEM_DIM), jnp.float32),
        pltpu.SemaphoreType.DMA,
    ],
)
def _sc_scatter(mem_ref, hnew_hbm, idx_hbm, idx_v, rows_v, sem):
    wid = _wid()
    base = wid * B_PER_W
    pltpu.sync_copy(idx_hbm.at[pl.ds(wid * N_CHUNKS, N_CHUNKS)], idx_v)
    pltpu.sync_copy(hnew_hbm.at[pl.ds(base, B_PER_W)], rows_v)
    copies = []
    for j in range(N_CHUNKS):
        copies.append(
            pltpu.async_copy(
                rows_v.at[pl.ds(j * CHUNK, CHUNK)],
                mem_ref.at[idx_v.at[j]],
                sem,
            )
        )
    for c in copies:
        c.wait()


def kernel(unique_nids, unique_msg, time, memory, last_update,
           W_ih, W_hh, b_ih, b_hh):
    idx = unique_nids.astype(jnp.int32)
    idx2d = jnp.reshape(idx, (NW * N_CHUNKS, CHUNK))
    tvals = jnp.full((LANES,), time, dtype=jnp.float32)
    mem_ref = jax.new_ref(memory)
    h, lu_out = _sc_gather(memory, idx, tvals, last_update)
    h_new = _tc_gru(
        unique_msg, h,
        W_ih.T, W_hh.T,
        b_ih.reshape(1, -1), b_hh.reshape(1, -1),
    )
    _sc_scatter(mem_ref, h_new, idx2d)
    return mem_ref[...], lu_out


# speedup vs baseline: 16.4910x; 1.0108x over previous
import functools

import jax
import jax.numpy as jnp
from jax import lax
from jax.experimental import pallas as pl
from jax.experimental.pallas import tpu as pltpu
from jax.experimental.pallas import tpu_sc as plsc

N_NODES = 100000
MEM_DIM = 128
MSG_DIM = 256
B = 16384

NC = 2
NS = 16
NW = NC * NS
B_PER_W = B // NW
CHUNK = 128
N_CHUNKS = B_PER_W // CHUNK
LANES = 16
LU_PER_W = 3128

_MESH = plsc.VectorSubcoreMesh(
    core_axis_name="c", subcore_axis_name="s", num_cores=NC, num_subcores=NS
)


def _wid():
    return lax.axis_index("s") * NC + lax.axis_index("c")


@functools.partial(
    pl.kernel,
    mesh=_MESH,
    out_type=(
        jax.ShapeDtypeStruct((B, MEM_DIM), jnp.float32),
        jax.ShapeDtypeStruct((N_NODES,), jnp.float32),
    ),
    scratch_types=[
        pltpu.VMEM((B_PER_W,), jnp.int32),
        pltpu.VMEM((B,), jnp.int32),
        pltpu.VMEM((B_PER_W, MEM_DIM), jnp.float32),
        pltpu.VMEM((LU_PER_W,), jnp.float32),
        pltpu.VMEM((LANES,), jnp.float32),
        pltpu.SemaphoreType.DMA,
    ],
    compiler_params=pltpu.CompilerParams(needs_layout_passes=False),
)
def _sc_gather(mem_hbm, idx_hbm, tvals_hbm, lu_hbm, h_hbm, lu_out_hbm,
               idx_v, idxf_v, rows_v, lu_v, tv_v, sem):
    wid = _wid()
    base = wid * B_PER_W
    pltpu.sync_copy(idx_hbm.at[pl.ds(base, B_PER_W)], idx_v)
    copies = []
    for j in range(N_CHUNKS):
        copies.append(
            pltpu.async_copy(
                mem_hbm.at[idx_v.at[pl.ds(j * CHUNK, CHUNK)]],
                rows_v.at[pl.ds(j * CHUNK, CHUNK)],
                sem,
            )
        )
    lub = jnp.minimum(wid * LU_PER_W, N_NODES - LU_PER_W)
    pltpu.sync_copy(idx_hbm, idxf_v)
    pltpu.sync_copy(lu_hbm.at[pl.ds(lub, LU_PER_W)], lu_v)
    pltpu.sync_copy(tvals_hbm, tv_v)
    tvec = tv_v[...]

    _UNROLL = 8

    def _scan_body(i, tv):
        for k in range(_UNROLL):
            v = idxf_v[pl.ds((i * _UNROLL + k) * LANES, LANES)]
            rel = v - lub
            m = (rel >= 0) & (rel < LU_PER_W)
            plsc.store_scatter(lu_v, [rel], tv, mask=m)
        return tv

    lax.fori_loop(0, B // (LANES * _UNROLL), _scan_body, tvec)
    pltpu.sync_copy(lu_v, lu_out_hbm.at[pl.ds(lub, LU_PER_W)])
    for c in copies:
        c.wait()
    pltpu.sync_copy(rows_v, h_hbm.at[pl.ds(base, B_PER_W)])


_BLK = 2048


def _gru_body(x_ref, h_ref, wi_ref, wh_ref, bi_ref, bh_ref, o_ref):
    h = h_ref[...]
    gi = jnp.dot(x_ref[...], wi_ref[...], preferred_element_type=jnp.float32)
    gh = jnp.dot(h, wh_ref[...], preferred_element_type=jnp.float32)
    gi = gi + bi_ref[...]
    gh = gh + bh_ref[...]
    r = jax.nn.sigmoid(gi[:, :MEM_DIM] + gh[:, :MEM_DIM])
    z = jax.nn.sigmoid(gi[:, MEM_DIM:2 * MEM_DIM] + gh[:, MEM_DIM:2 * MEM_DIM])
    n = jnp.tanh(gi[:, 2 * MEM_DIM:] + r * gh[:, 2 * MEM_DIM:])
    o_ref[...] = (1.0 - z) * n + z * h


def _tc_gru(x, h, wi_t, wh_t, bi, bh):
    grid = (B // _BLK,)
    return pl.pallas_call(
        _gru_body,
        grid=grid,
        in_specs=[
            pl.BlockSpec((_BLK, MSG_DIM), lambda i: (i, 0)),
            pl.BlockSpec((_BLK, MEM_DIM), lambda i: (i, 0)),
            pl.BlockSpec((MSG_DIM, 3 * MEM_DIM), lambda i: (0, 0)),
            pl.BlockSpec((MEM_DIM, 3 * MEM_DIM), lambda i: (0, 0)),
            pl.BlockSpec((1, 3 * MEM_DIM), lambda i: (0, 0)),
            pl.BlockSpec((1, 3 * MEM_DIM), lambda i: (0, 0)),
        ],
        out_specs=pl.BlockSpec((_BLK, MEM_DIM), lambda i: (i, 0)),
        out_shape=jax.ShapeDtypeStruct((B, MEM_DIM), jnp.float32),
    )(x, h, wi_t, wh_t, bi, bh)


@functools.partial(
    pl.kernel,
    mesh=_MESH,
    out_type=(),
    scratch_types=[
        pltpu.VMEM((N_CHUNKS, CHUNK), jnp.int32),
        pltpu.VMEM((B_PER_W, MEM_DIM), jnp.float32),
        pltpu.SemaphoreType.DMA,
        pltpu.SemaphoreType.DMA,
    ],
)
def _sc_scatter(mem_ref, hnew_hbm, idx_hbm, idx_v, rows_v, sem, sem_in):
    wid = _wid()
    base = wid * B_PER_W
    pltpu.sync_copy(idx_hbm.at[pl.ds(wid * N_CHUNKS, N_CHUNKS)], idx_v)
    stage = []
    for j in range(N_CHUNKS):
        stage.append(
            pltpu.async_copy(
                hnew_hbm.at[pl.ds(base + j * CHUNK, CHUNK)],
                rows_v.at[pl.ds(j * CHUNK, CHUNK)],
                sem_in,
            )
        )
    copies = []
    for j in range(N_CHUNKS):
        stage[j].wait()
        copies.append(
            pltpu.async_copy(
                rows_v.at[pl.ds(j * CHUNK, CHUNK)],
                mem_ref.at[idx_v.at[j]],
                sem,
            )
        )
    for c in copies:
        c.wait()


def kernel(unique_nids, unique_msg, time, memory, last_update,
           W_ih, W_hh, b_ih, b_hh):
    idx = unique_nids.astype(jnp.int32)
    idx2d = jnp.reshape(idx, (NW * N_CHUNKS, CHUNK))
    tvals = jnp.full((LANES,), time, dtype=jnp.float32)
    mem_ref = jax.new_ref(memory)
    h, lu_out = _sc_gather(memory, idx, tvals, last_update)
    h_new = _tc_gru(
        unique_msg, h,
        W_ih.T, W_hh.T,
        b_ih.reshape(1, -1), b_hh.reshape(1, -1),
    )
    _sc_scatter(mem_ref, h_new, idx2d)
    return mem_ref[...], lu_out


# speedup vs baseline: 16.7653x; 1.0166x over previous
import functools

import jax
import jax.numpy as jnp
from jax import lax
from jax.experimental import pallas as pl
from jax.experimental.pallas import tpu as pltpu
from jax.experimental.pallas import tpu_sc as plsc

N_NODES = 100000
MEM_DIM = 128
MSG_DIM = 256
B = 16384

NC = 2
NS = 16
NW = NC * NS
B_PER_W = B // NW
CHUNK = 128
N_CHUNKS = B_PER_W // CHUNK
LANES = 16
LU_PER_W = 3128

_MESH = plsc.VectorSubcoreMesh(
    core_axis_name="c", subcore_axis_name="s", num_cores=NC, num_subcores=NS
)


def _wid():
    return lax.axis_index("s") * NC + lax.axis_index("c")


@functools.partial(
    pl.kernel,
    mesh=_MESH,
    out_type=(
        jax.ShapeDtypeStruct((B, MEM_DIM), jnp.float32),
        jax.ShapeDtypeStruct((N_NODES,), jnp.float32),
    ),
    scratch_types=[
        pltpu.VMEM((B_PER_W,), jnp.int32),
        pltpu.VMEM((B,), jnp.int32),
        pltpu.VMEM((B_PER_W, MEM_DIM), jnp.float32),
        pltpu.VMEM((LU_PER_W,), jnp.float32),
        pltpu.VMEM((LANES,), jnp.float32),
        pltpu.SemaphoreType.DMA,
        pltpu.SemaphoreType.DMA,
    ],
    compiler_params=pltpu.CompilerParams(needs_layout_passes=False),
)
def _sc_gather(mem_hbm, idx_hbm, tvals_hbm, lu_hbm, h_hbm, lu_out_hbm,
               idx_v, idxf_v, rows_v, lu_v, tv_v, sem, sem_in):
    wid = _wid()
    base = wid * B_PER_W
    pltpu.sync_copy(idx_hbm.at[pl.ds(base, B_PER_W)], idx_v)
    copies = []
    for j in range(N_CHUNKS):
        copies.append(
            pltpu.async_copy(
                mem_hbm.at[idx_v.at[pl.ds(j * CHUNK, CHUNK)]],
                rows_v.at[pl.ds(j * CHUNK, CHUNK)],
                sem,
            )
        )
    idxf_cp = pltpu.async_copy(idx_hbm, idxf_v, sem_in)
    lub = jnp.minimum(wid * LU_PER_W, N_NODES - LU_PER_W)
    pltpu.sync_copy(lu_hbm.at[pl.ds(lub, LU_PER_W)], lu_v)
    pltpu.sync_copy(tvals_hbm, tv_v)
    tvec = tv_v[...]
    idxf_cp.wait()

    _UNROLL = 8
    _PART = B // (LANES * _UNROLL * N_CHUNKS)

    def _scan_body(i, tv):
        for k in range(_UNROLL):
            v = idxf_v[pl.ds((i * _UNROLL + k) * LANES, LANES)]
            rel = v - lub
            m = (rel >= 0) & (rel < LU_PER_W)
            plsc.store_scatter(lu_v, [rel], tv, mask=m)
        return tv

    outs = []
    for j in range(N_CHUNKS):
        lax.fori_loop(j * _PART, (j + 1) * _PART, _scan_body, tvec)
        copies[j].wait()
        outs.append(
            pltpu.async_copy(
                rows_v.at[pl.ds(j * CHUNK, CHUNK)],
                h_hbm.at[pl.ds(base + j * CHUNK, CHUNK)],
                sem,
            )
        )
    pltpu.sync_copy(lu_v, lu_out_hbm.at[pl.ds(lub, LU_PER_W)])
    for o in outs:
        o.wait()


_BLK = 2048


def _gru_body(x_ref, h_ref, wi_ref, wh_ref, bi_ref, bh_ref, o_ref):
    h = h_ref[...]
    gi = jnp.dot(x_ref[...], wi_ref[...], preferred_element_type=jnp.float32)
    gh = jnp.dot(h, wh_ref[...], preferred_element_type=jnp.float32)
    gi = gi + bi_ref[...]
    gh = gh + bh_ref[...]
    r = jax.nn.sigmoid(gi[:, :MEM_DIM] + gh[:, :MEM_DIM])
    z = jax.nn.sigmoid(gi[:, MEM_DIM:2 * MEM_DIM] + gh[:, MEM_DIM:2 * MEM_DIM])
    n = jnp.tanh(gi[:, 2 * MEM_DIM:] + r * gh[:, 2 * MEM_DIM:])
    o_ref[...] = (1.0 - z) * n + z * h


def _tc_gru(x, h, wi_t, wh_t, bi, bh):
    grid = (B // _BLK,)
    return pl.pallas_call(
        _gru_body,
        grid=grid,
        in_specs=[
            pl.BlockSpec((_BLK, MSG_DIM), lambda i: (i, 0)),
            pl.BlockSpec((_BLK, MEM_DIM), lambda i: (i, 0)),
            pl.BlockSpec((MSG_DIM, 3 * MEM_DIM), lambda i: (0, 0)),
            pl.BlockSpec((MEM_DIM, 3 * MEM_DIM), lambda i: (0, 0)),
            pl.BlockSpec((1, 3 * MEM_DIM), lambda i: (0, 0)),
            pl.BlockSpec((1, 3 * MEM_DIM), lambda i: (0, 0)),
        ],
        out_specs=pl.BlockSpec((_BLK, MEM_DIM), lambda i: (i, 0)),
        out_shape=jax.ShapeDtypeStruct((B, MEM_DIM), jnp.float32),
    )(x, h, wi_t, wh_t, bi, bh)


@functools.partial(
    pl.kernel,
    mesh=_MESH,
    out_type=(),
    scratch_types=[
        pltpu.VMEM((N_CHUNKS, CHUNK), jnp.int32),
        pltpu.VMEM((B_PER_W, MEM_DIM), jnp.float32),
        pltpu.SemaphoreType.DMA,
        pltpu.SemaphoreType.DMA,
    ],
)
def _sc_scatter(mem_ref, hnew_hbm, idx_hbm, idx_v, rows_v, sem, sem_in):
    wid = _wid()
    base = wid * B_PER_W
    pltpu.sync_copy(idx_hbm.at[pl.ds(wid * N_CHUNKS, N_CHUNKS)], idx_v)
    stage = []
    for j in range(N_CHUNKS):
        stage.append(
            pltpu.async_copy(
                hnew_hbm.at[pl.ds(base + j * CHUNK, CHUNK)],
                rows_v.at[pl.ds(j * CHUNK, CHUNK)],
                sem_in,
            )
        )
    copies = []
    for j in range(N_CHUNKS):
        stage[j].wait()
        copies.append(
            pltpu.async_copy(
                rows_v.at[pl.ds(j * CHUNK, CHUNK)],
                mem_ref.at[idx_v.at[j]],
                sem,
            )
        )
    for c in copies:
        c.wait()


def kernel(unique_nids, unique_msg, time, memory, last_update,
           W_ih, W_hh, b_ih, b_hh):
    idx = unique_nids.astype(jnp.int32)
    idx2d = jnp.reshape(idx, (NW * N_CHUNKS, CHUNK))
    tvals = jnp.full((LANES,), time, dtype=jnp.float32)
    mem_ref = jax.new_ref(memory)
    h, lu_out = _sc_gather(memory, idx, tvals, last_update)
    h_new = _tc_gru(
        unique_msg, h,
        W_ih.T, W_hh.T,
        b_ih.reshape(1, -1), b_hh.reshape(1, -1),
    )
    _sc_scatter(mem_ref, h_new, idx2d)
    return mem_ref[...], lu_out


# speedup vs baseline: 17.0136x; 1.0148x over previous
import functools

import jax
import jax.numpy as jnp
from jax import lax
from jax.experimental import pallas as pl
from jax.experimental.pallas import tpu as pltpu
from jax.experimental.pallas import tpu_sc as plsc

N_NODES = 100000
MEM_DIM = 128
MSG_DIM = 256
B = 16384

NC = 2
NS = 16
NW = NC * NS
B_PER_W = B // NW
CHUNK = 128
N_CHUNKS = B_PER_W // CHUNK
LANES = 16
LU_PER_W = 3128

_MESH = plsc.VectorSubcoreMesh(
    core_axis_name="c", subcore_axis_name="s", num_cores=NC, num_subcores=NS
)


def _wid():
    return lax.axis_index("s") * NC + lax.axis_index("c")


@functools.partial(
    pl.kernel,
    mesh=_MESH,
    out_type=(
        jax.ShapeDtypeStruct((B, MEM_DIM), jnp.float32),
        jax.ShapeDtypeStruct((N_NODES,), jnp.float32),
    ),
    scratch_types=[
        pltpu.VMEM((B_PER_W,), jnp.int32),
        pltpu.VMEM((B,), jnp.int32),
        pltpu.VMEM((B_PER_W, MEM_DIM), jnp.float32),
        pltpu.VMEM((LU_PER_W,), jnp.float32),
        pltpu.VMEM((LANES,), jnp.float32),
        pltpu.SemaphoreType.DMA,
        pltpu.SemaphoreType.DMA,
    ],
    compiler_params=pltpu.CompilerParams(needs_layout_passes=False),
)
def _sc_gather(mem_hbm, idx_hbm, tvals_hbm, lu_hbm, h_hbm, lu_out_hbm,
               idx_v, idxf_v, rows_v, lu_v, tv_v, sem, sem_in):
    wid = _wid()
    base = wid * B_PER_W
    pltpu.sync_copy(idx_hbm.at[pl.ds(base, B_PER_W)], idx_v)
    copies = []
    for j in range(N_CHUNKS):
        copies.append(
            pltpu.async_copy(
                mem_hbm.at[idx_v.at[pl.ds(j * CHUNK, CHUNK)]],
                rows_v.at[pl.ds(j * CHUNK, CHUNK)],
                sem,
            )
        )
    idxf_cp = pltpu.async_copy(idx_hbm, idxf_v, sem_in)
    lub = jnp.minimum(wid * LU_PER_W, N_NODES - LU_PER_W)
    pltpu.sync_copy(lu_hbm.at[pl.ds(lub, LU_PER_W)], lu_v)
    pltpu.sync_copy(tvals_hbm, tv_v)
    tvec = tv_v[...]
    idxf_cp.wait()

    _UNROLL = 8
    _PART = B // (LANES * _UNROLL * N_CHUNKS)

    def _scan_body(i, tv):
        for k in range(_UNROLL):
            v = idxf_v[pl.ds((i * _UNROLL + k) * LANES, LANES)]
            rel = v - lub
            m = (rel >= 0) & (rel < LU_PER_W)
            plsc.store_scatter(lu_v, [rel], tv, mask=m)
        return tv

    outs = []
    for j in range(N_CHUNKS):
        lax.fori_loop(j * _PART, (j + 1) * _PART, _scan_body, tvec)
        copies[j].wait()
        outs.append(
            pltpu.async_copy(
                rows_v.at[pl.ds(j * CHUNK, CHUNK)],
                h_hbm.at[pl.ds(base + j * CHUNK, CHUNK)],
                sem,
            )
        )
    pltpu.sync_copy(lu_v, lu_out_hbm.at[pl.ds(lub, LU_PER_W)])
    for o in outs:
        o.wait()


_BLK = 4096


def _gru_body(x_ref, h_ref, wi_ref, wh_ref, bi_ref, bh_ref, o_ref):
    h = h_ref[...]
    gi = jnp.dot(x_ref[...], wi_ref[...], preferred_element_type=jnp.float32)
    gh = jnp.dot(h, wh_ref[...], preferred_element_type=jnp.float32)
    gi = gi + bi_ref[...]
    gh = gh + bh_ref[...]
    r = jax.nn.sigmoid(gi[:, :MEM_DIM] + gh[:, :MEM_DIM])
    z = jax.nn.sigmoid(gi[:, MEM_DIM:2 * MEM_DIM] + gh[:, MEM_DIM:2 * MEM_DIM])
    n = jnp.tanh(gi[:, 2 * MEM_DIM:] + r * gh[:, 2 * MEM_DIM:])
    o_ref[...] = (1.0 - z) * n + z * h


def _tc_gru(x, h, wi_t, wh_t, bi, bh):
    grid = (B // _BLK,)
    return pl.pallas_call(
        _gru_body,
        grid=grid,
        in_specs=[
            pl.BlockSpec((_BLK, MSG_DIM), lambda i: (i, 0)),
            pl.BlockSpec((_BLK, MEM_DIM), lambda i: (i, 0)),
            pl.BlockSpec((MSG_DIM, 3 * MEM_DIM), lambda i: (0, 0)),
            pl.BlockSpec((MEM_DIM, 3 * MEM_DIM), lambda i: (0, 0)),
            pl.BlockSpec((1, 3 * MEM_DIM), lambda i: (0, 0)),
            pl.BlockSpec((1, 3 * MEM_DIM), lambda i: (0, 0)),
        ],
        out_specs=pl.BlockSpec((_BLK, MEM_DIM), lambda i: (i, 0)),
        out_shape=jax.ShapeDtypeStruct((B, MEM_DIM), jnp.float32),
    )(x, h, wi_t, wh_t, bi, bh)


@functools.partial(
    pl.kernel,
    mesh=_MESH,
    out_type=(),
    scratch_types=[
        pltpu.VMEM((N_CHUNKS, CHUNK), jnp.int32),
        pltpu.VMEM((B_PER_W, MEM_DIM), jnp.float32),
        pltpu.SemaphoreType.DMA,
        pltpu.SemaphoreType.DMA,
    ],
)
def _sc_scatter(mem_ref, hnew_hbm, idx_hbm, idx_v, rows_v, sem, sem_in):
    wid = _wid()
    base = wid * B_PER_W
    pltpu.sync_copy(idx_hbm.at[pl.ds(wid * N_CHUNKS, N_CHUNKS)], idx_v)
    stage = []
    for j in range(N_CHUNKS):
        stage.append(
            pltpu.async_copy(
                hnew_hbm.at[pl.ds(base + j * CHUNK, CHUNK)],
                rows_v.at[pl.ds(j * CHUNK, CHUNK)],
                sem_in,
            )
        )
    copies = []
    for j in range(N_CHUNKS):
        stage[j].wait()
        copies.append(
            pltpu.async_copy(
                rows_v.at[pl.ds(j * CHUNK, CHUNK)],
                mem_ref.at[idx_v.at[j]],
                sem,
            )
        )
    for c in copies:
        c.wait()


def kernel(unique_nids, unique_msg, time, memory, last_update,
           W_ih, W_hh, b_ih, b_hh):
    idx = unique_nids.astype(jnp.int32)
    idx2d = jnp.reshape(idx, (NW * N_CHUNKS, CHUNK))
    tvals = jnp.full((LANES,), time, dtype=jnp.float32)
    mem_ref = jax.new_ref(memory)
    h, lu_out = _sc_gather(memory, idx, tvals, last_update)
    h_new = _tc_gru(
        unique_msg, h,
        W_ih.T, W_hh.T,
        b_ih.reshape(1, -1), b_hh.reshape(1, -1),
    )
    _sc_scatter(mem_ref, h_new, idx2d)
    return mem_ref[...], lu_out


# speedup vs baseline: 17.2211x; 1.0122x over previous
import functools

import jax
import jax.numpy as jnp
from jax import lax
from jax.experimental import pallas as pl
from jax.experimental.pallas import tpu as pltpu
from jax.experimental.pallas import tpu_sc as plsc

N_NODES = 100000
MEM_DIM = 128
MSG_DIM = 256
B = 16384

NC = 2
NS = 16
NW = NC * NS
B_PER_W = B // NW
CHUNK = 128
N_CHUNKS = B_PER_W // CHUNK
LANES = 16
LU_PER_W = 3128

_MESH = plsc.VectorSubcoreMesh(
    core_axis_name="c", subcore_axis_name="s", num_cores=NC, num_subcores=NS
)


def _wid():
    return lax.axis_index("s") * NC + lax.axis_index("c")


@functools.partial(
    pl.kernel,
    mesh=_MESH,
    out_type=(
        jax.ShapeDtypeStruct((B, MEM_DIM), jnp.float32),
        jax.ShapeDtypeStruct((N_NODES,), jnp.float32),
    ),
    scratch_types=[
        pltpu.VMEM((B_PER_W,), jnp.int32),
        pltpu.VMEM((B,), jnp.int32),
        pltpu.VMEM((B_PER_W, MEM_DIM), jnp.float32),
        pltpu.VMEM((LU_PER_W,), jnp.float32),
        pltpu.VMEM((LANES,), jnp.float32),
        pltpu.SemaphoreType.DMA,
        pltpu.SemaphoreType.DMA,
    ],
    compiler_params=pltpu.CompilerParams(needs_layout_passes=False),
)
def _sc_gather(mem_hbm, idx_hbm, tvals_hbm, lu_hbm, h_hbm, lu_out_hbm,
               idx_v, idxf_v, rows_v, lu_v, tv_v, sem, sem_in):
    wid = _wid()
    base = wid * B_PER_W
    pltpu.sync_copy(idx_hbm.at[pl.ds(base, B_PER_W)], idx_v)
    copies = []
    for j in range(N_CHUNKS):
        copies.append(
            pltpu.async_copy(
                mem_hbm.at[idx_v.at[pl.ds(j * CHUNK, CHUNK)]],
                rows_v.at[pl.ds(j * CHUNK, CHUNK)],
                sem,
            )
        )
    idxf_cps = []
    for k in range(NW):
        src = ((wid + k) % NW) * B_PER_W
        idxf_cps.append(
            pltpu.async_copy(
                idx_hbm.at[pl.ds(src, B_PER_W)],
                idxf_v.at[pl.ds(k * B_PER_W, B_PER_W)],
                sem_in,
            )
        )
    lub = jnp.minimum(wid * LU_PER_W, N_NODES - LU_PER_W)
    pltpu.sync_copy(lu_hbm.at[pl.ds(lub, LU_PER_W)], lu_v)
    pltpu.sync_copy(tvals_hbm, tv_v)
    tvec = tv_v[...]
    for c in idxf_cps:
        c.wait()

    _UNROLL = 8
    _PART = B // (LANES * _UNROLL * N_CHUNKS)

    def _scan_body(i, tv):
        for k in range(_UNROLL):
            v = idxf_v[pl.ds((i * _UNROLL + k) * LANES, LANES)]
            rel = v - lub
            m = (rel >= 0) & (rel < LU_PER_W)
            plsc.store_scatter(lu_v, [rel], tv, mask=m)
        return tv

    outs = []
    for j in range(N_CHUNKS):
        lax.fori_loop(j * _PART, (j + 1) * _PART, _scan_body, tvec)
        copies[j].wait()
        outs.append(
            pltpu.async_copy(
                rows_v.at[pl.ds(j * CHUNK, CHUNK)],
                h_hbm.at[pl.ds(base + j * CHUNK, CHUNK)],
                sem,
            )
        )
    pltpu.sync_copy(lu_v, lu_out_hbm.at[pl.ds(lub, LU_PER_W)])
    for o in outs:
        o.wait()


_BLK = 4096


def _gru_body(x_ref, h_ref, wi_ref, wh_ref, bi_ref, bh_ref, o_ref):
    h = h_ref[...]
    gi = jnp.dot(x_ref[...], wi_ref[...], preferred_element_type=jnp.float32)
    gh = jnp.dot(h, wh_ref[...], preferred_element_type=jnp.float32)
    gi = gi + bi_ref[...]
    gh = gh + bh_ref[...]
    r = jax.nn.sigmoid(gi[:, :MEM_DIM] + gh[:, :MEM_DIM])
    z = jax.nn.sigmoid(gi[:, MEM_DIM:2 * MEM_DIM] + gh[:, MEM_DIM:2 * MEM_DIM])
    n = jnp.tanh(gi[:, 2 * MEM_DIM:] + r * gh[:, 2 * MEM_DIM:])
    o_ref[...] = (1.0 - z) * n + z * h


def _tc_gru(x, h, wi_t, wh_t, bi, bh):
    grid = (B // _BLK,)
    return pl.pallas_call(
        _gru_body,
        grid=grid,
        in_specs=[
            pl.BlockSpec((_BLK, MSG_DIM), lambda i: (i, 0)),
            pl.BlockSpec((_BLK, MEM_DIM), lambda i: (i, 0)),
            pl.BlockSpec((MSG_DIM, 3 * MEM_DIM), lambda i: (0, 0)),
            pl.BlockSpec((MEM_DIM, 3 * MEM_DIM), lambda i: (0, 0)),
            pl.BlockSpec((1, 3 * MEM_DIM), lambda i: (0, 0)),
            pl.BlockSpec((1, 3 * MEM_DIM), lambda i: (0, 0)),
        ],
        out_specs=pl.BlockSpec((_BLK, MEM_DIM), lambda i: (i, 0)),
        out_shape=jax.ShapeDtypeStruct((B, MEM_DIM), jnp.float32),
    )(x, h, wi_t, wh_t, bi, bh)


@functools.partial(
    pl.kernel,
    mesh=_MESH,
    out_type=(),
    scratch_types=[
        pltpu.VMEM((N_CHUNKS, CHUNK), jnp.int32),
        pltpu.VMEM((B_PER_W, MEM_DIM), jnp.float32),
        pltpu.SemaphoreType.DMA,
        pltpu.SemaphoreType.DMA,
    ],
)
def _sc_scatter(mem_ref, hnew_hbm, idx_hbm, idx_v, rows_v, sem, sem_in):
    wid = _wid()
    base = wid * B_PER_W
    pltpu.sync_copy(idx_hbm.at[pl.ds(wid * N_CHUNKS, N_CHUNKS)], idx_v)
    stage = []
    for j in range(N_CHUNKS):
        stage.append(
            pltpu.async_copy(
                hnew_hbm.at[pl.ds(base + j * CHUNK, CHUNK)],
                rows_v.at[pl.ds(j * CHUNK, CHUNK)],
                sem_in,
            )
        )
    copies = []
    for j in range(N_CHUNKS):
        stage[j].wait()
        copies.append(
            pltpu.async_copy(
                rows_v.at[pl.ds(j * CHUNK, CHUNK)],
                mem_ref.at[idx_v.at[j]],
                sem,
            )
        )
    for c in copies:
        c.wait()


def kernel(unique_nids, unique_msg, time, memory, last_update,
           W_ih, W_hh, b_ih, b_hh):
    idx = unique_nids.astype(jnp.int32)
    idx2d = jnp.reshape(idx, (NW * N_CHUNKS, CHUNK))
    tvals = jnp.full((LANES,), time, dtype=jnp.float32)
    mem_ref = jax.new_ref(memory)
    h, lu_out = _sc_gather(memory, idx, tvals, last_update)
    h_new = _tc_gru(
        unique_msg, h,
        W_ih.T, W_hh.T,
        b_ih.reshape(1, -1), b_hh.reshape(1, -1),
    )
    _sc_scatter(mem_ref, h_new, idx2d)
    return mem_ref[...], lu_out
